# Initial kernel scaffold; baseline (speedup 1.0000x reference)
import jax
import jax.numpy as jnp
from jax.experimental import pallas as pl


def kernel(x, edge_index, W1_l, b1, W1_r, W2_l, b2, W2_r):
    raise NotImplementedError("write your pallas kernel here")


# speedup vs baseline: 13.5667x; 13.5667x over previous
import functools

import jax
import jax.numpy as jnp
from jax import lax
from jax.experimental import pallas as pl
from jax.experimental.pallas import tpu as pltpu
from jax.experimental.pallas import tpu_sc as plsc

NC = 2
NS = 16
CH = 128


def _seg_sum_sc(table, srcp, dstp, zrows, zdeg, n_pad, width, with_deg, k):
    nw = NC * NS
    chunks = srcp.shape[1]
    rpt = n_pad // NS
    mesh = plsc.VectorSubcoreMesh(core_axis_name="c", subcore_axis_name="s")

    out_type = [jax.ShapeDtypeStruct((NC, n_pad, width), jnp.float32)]
    if with_deg:
        out_type.append(jax.ShapeDtypeStruct((NC, n_pad), jnp.float32))

    scratch = [
        pltpu.VMEM((chunks, CH), jnp.int32),
        pltpu.VMEM((chunks, CH), jnp.int32),
        pltpu.VMEM((k, CH, width), jnp.float32),
        pltpu.VMEM_SHARED((n_pad, width), jnp.float32),
        pltpu.SemaphoreType.DMA,
        pltpu.SemaphoreType.DMA,
    ]
    if with_deg:
        scratch += [
            pltpu.VMEM((CH,), jnp.float32),
            pltpu.VMEM_SHARED((n_pad,), jnp.float32),
        ]

    def body(table_h, src_h, dst_h, zrows_h, *rest):
        if with_deg:
            (zdeg_h, acc_out, deg_out,
             src_v, dst_v, rows, acc, sem_g, sem_s, ones, dacc) = rest
        else:
            (acc_out, src_v, dst_v, rows, acc, sem_g, sem_s) = rest
        c = lax.axis_index("c")
        s = lax.axis_index("s")
        wid = s * NC + c
        tile_rows = pl.ds(s * rpt, rpt)
        pltpu.sync_copy(zrows_h.at[tile_rows], acc.at[tile_rows])
        pltpu.sync_copy(src_h.at[wid], src_v)
        pltpu.sync_copy(dst_h.at[wid], dst_v)
        if with_deg:
            pltpu.sync_copy(zdeg_h.at[tile_rows], dacc.at[tile_rows])
            for i in range(CH // 16):
                ones[pl.ds(i * 16, 16)] = jnp.full((16,), 1.0, jnp.float32)
        plsc.subcore_barrier()

        def superchunk(j, carry):
            jk = j * k
            gd = [
                pltpu.async_copy(
                    table_h.at[src_v.at[jk + i]], rows.at[i], sem_g)
                for i in range(k)
            ]
            for d in gd:
                d.wait()
            sd = []
            for i in range(k):
                sd.append(pltpu.async_copy(
                    rows.at[i], acc.at[dst_v.at[jk + i]], sem_s, add=True))
                if with_deg:
                    sd.append(pltpu.async_copy(
                        ones, dacc.at[dst_v.at[jk + i]], sem_s, add=True))
            for d in sd:
                d.wait()
            return carry

        lax.fori_loop(0, chunks // k, superchunk, 0)
        plsc.subcore_barrier()
        pltpu.sync_copy(acc.at[tile_rows], acc_out.at[c].at[tile_rows])
        if with_deg:
            pltpu.sync_copy(dacc.at[tile_rows], deg_out.at[c].at[tile_rows])

    fn = pl.kernel(
        body, out_type=out_type, mesh=mesh, scratch_types=scratch,
        compiler_params=pltpu.CompilerParams(use_tc_tiling_on_sc=False))
    if with_deg:
        return fn(table, srcp, dstp, zrows, zdeg)
    return fn(table, srcp, dstp, zrows)


def _tc1(x, w1cT, n, h):
    def body(x_ref, w_ref, y1_ref, r1_ref):
        t = jnp.dot(x_ref[...], w_ref[...], preferred_element_type=jnp.float32)
        y1_ref[...] = t[:, :h]
        r1_ref[...] = t[:, h:]

    return pl.pallas_call(
        body,
        out_shape=[
            jax.ShapeDtypeStruct((n, h), jnp.float32),
            jax.ShapeDtypeStruct((n, h), jnp.float32),
        ],
    )(x, w1cT)


def _tc2(s1p, degp, r1, b1, w2cT, bias8, n, n_pad, tw):
    def body(s1p_ref, degp_ref, r1_ref, b1_ref, w_ref, b4_ref, t2_ref, inv_ref):
        deg = jnp.maximum(degp_ref[0] + degp_ref[1], 1.0)
        inv = (1.0 / deg)[:, None]
        z = (s1p_ref[0] + s1p_ref[1]) * inv
        hh = jax.nn.relu(z[:n] + r1_ref[...] + b1_ref[...])
        t2_ref[...] = (
            jnp.dot(hh, w_ref[...], preferred_element_type=jnp.float32)
            + b4_ref[...]
        )
        inv_ref[...] = inv[:n]

    return pl.pallas_call(
        body,
        out_shape=[
            jax.ShapeDtypeStruct((n, tw), jnp.float32),
            jax.ShapeDtypeStruct((n, 1), jnp.float32),
        ],
    )(s1p, degp, r1, b1, w2cT, bias8)


def _tc3(s2p, t2, inv, n):
    def body(s2p_ref, t2_ref, inv_ref, out_ref):
        s2 = s2p_ref[0, :n, :2] + s2p_ref[1, :n, :2]
        out_ref[...] = s2 * inv_ref[...] + t2_ref[:, 2:4]

    return pl.pallas_call(
        body,
        out_shape=jax.ShapeDtypeStruct((n, 2), jnp.float32),
    )(s2p, t2, inv)


def kernel(x, edge_index, W1_l, b1, W1_r, W2_l, b2, W2_r):
    n, d = x.shape
    h = W1_l.shape[0]
    o = W2_l.shape[0]
    e = edge_index.shape[1]

    nw = NC * NS
    k1, k2 = 5, 8
    kq = k1 * k2
    step = nw * CH * kq
    e_pad = ((e + step - 1) // step) * step
    chunks = e_pad // (nw * CH)
    n_pad = ((n + 128 * NS - 1) // (128 * NS)) * (128 * NS)

    srcp = jnp.concatenate(
        [edge_index[0], jnp.zeros((e_pad - e,), jnp.int32)]
    ).reshape(nw, chunks, CH)
    pad_dst = n + jnp.arange(e_pad - e, dtype=jnp.int32) % (n_pad - n)
    dstp = jnp.concatenate([edge_index[1], pad_dst]).reshape(nw, chunks, CH)

    w1cT = jnp.concatenate([W1_l, W1_r], axis=0).T
    tw = 8
    w2cT = jnp.concatenate(
        [W2_l, W2_r, jnp.zeros((tw - 2 * o, W2_l.shape[1]), jnp.float32)],
        axis=0).T
    bias8 = jnp.concatenate(
        [jnp.zeros((o,), jnp.float32), b2,
         jnp.zeros((tw - 2 * o,), jnp.float32)])[None, :]

    z32 = jnp.zeros((n_pad, h), jnp.float32)
    z1 = jnp.zeros((n_pad,), jnp.float32)
    z8 = jnp.zeros((n_pad, tw), jnp.float32)

    y1, r1 = _tc1(x, w1cT, n, h)
    s1p, degp = _seg_sum_sc(y1, srcp, dstp, z32, z1, n_pad, h, True, k1)
    t2, inv = _tc2(s1p, degp, r1, b1[None, :], w2cT, bias8, n, n_pad, tw)
    s2p = _seg_sum_sc(t2, srcp, dstp, z8, None, n_pad, tw, False, k2)[0]
    out = _tc3(s2p, t2, inv, n)
    return out


# speedup vs baseline: 21.7969x; 1.6066x over previous
import functools

import jax
import jax.numpy as jnp
from jax import lax
from jax.experimental import pallas as pl
from jax.experimental.pallas import tpu as pltpu
from jax.experimental.pallas import tpu_sc as plsc

NC = 2
NS = 16
CH = 128


def _seg_sum_sc(table, srcp, dstp, zrows, zdeg, n_pad, width, with_deg, k):
    nw = NC * NS
    chunks = srcp.shape[1]
    rpt = n_pad // NS
    mesh = plsc.VectorSubcoreMesh(core_axis_name="c", subcore_axis_name="s")

    out_type = [jax.ShapeDtypeStruct((NC, n_pad, width), jnp.float32)]
    if with_deg:
        out_type.append(jax.ShapeDtypeStruct((NC, n_pad), jnp.float32))

    scratch = [
        pltpu.VMEM((chunks, CH), jnp.int32),
        pltpu.VMEM((chunks, CH), jnp.int32),
        pltpu.VMEM((k, CH, width), jnp.float32),
        pltpu.VMEM_SHARED((n_pad, width), jnp.float32),
        pltpu.VMEM_SHARED((n_pad, width), jnp.float32),
        pltpu.SemaphoreType.DMA,
        pltpu.SemaphoreType.DMA,
    ]
    if with_deg:
        scratch += [
            pltpu.VMEM((CH,), jnp.float32),
            pltpu.VMEM_SHARED((n_pad,), jnp.float32),
        ]

    def body(table_h, src_h, dst_h, zrows_h, *rest):
        if with_deg:
            (zdeg_h, acc_out, deg_out,
             src_v, dst_v, rows, acc, tbl, sem_g, sem_s, ones, dacc) = rest
        else:
            (acc_out, src_v, dst_v, rows, acc, tbl, sem_g, sem_s) = rest
        c = lax.axis_index("c")
        s = lax.axis_index("s")
        wid = s * NC + c
        tile_rows = pl.ds(s * rpt, rpt)
        pltpu.sync_copy(zrows_h.at[tile_rows], acc.at[tile_rows])
        pltpu.sync_copy(table_h.at[tile_rows], tbl.at[tile_rows])
        pltpu.sync_copy(src_h.at[wid], src_v)
        pltpu.sync_copy(dst_h.at[wid], dst_v)
        if with_deg:
            pltpu.sync_copy(zdeg_h.at[tile_rows], dacc.at[tile_rows])
            for i in range(CH // 16):
                ones[pl.ds(i * 16, 16)] = jnp.full((16,), 1.0, jnp.float32)
        plsc.subcore_barrier()

        def superchunk(j, carry):
            jk = j * k
            gd = [
                pltpu.async_copy(
                    tbl.at[src_v.at[jk + i]], rows.at[i], sem_g)
                for i in range(k)
            ]
            for d in gd:
                d.wait()
            sd = []
            for i in range(k):
                sd.append(pltpu.async_copy(
                    rows.at[i], acc.at[dst_v.at[jk + i]], sem_s, add=True))
                if with_deg:
                    sd.append(pltpu.async_copy(
                        ones, dacc.at[dst_v.at[jk + i]], sem_s, add=True))
            for d in sd:
                d.wait()
            return carry

        lax.fori_loop(0, chunks // k, superchunk, 0)
        plsc.subcore_barrier()
        pltpu.sync_copy(acc.at[tile_rows], acc_out.at[c].at[tile_rows])
        if with_deg:
            pltpu.sync_copy(dacc.at[tile_rows], deg_out.at[c].at[tile_rows])

    fn = pl.kernel(
        body, out_type=out_type, mesh=mesh, scratch_types=scratch,
        compiler_params=pltpu.CompilerParams(use_tc_tiling_on_sc=False))
    if with_deg:
        return fn(table, srcp, dstp, zrows, zdeg)
    return fn(table, srcp, dstp, zrows)


def _tc1(x, w1cT, n, n_pad, h):
    def body(x_ref, w_ref, y1_ref, r1_ref):
        t = jnp.dot(x_ref[...], w_ref[...], preferred_element_type=jnp.float32)
        y1_ref[...] = jnp.concatenate(
            [t[:, :h], jnp.zeros((n_pad - n, h), jnp.float32)])
        r1_ref[...] = t[:, h:]

    return pl.pallas_call(
        body,
        out_shape=[
            jax.ShapeDtypeStruct((n_pad, h), jnp.float32),
            jax.ShapeDtypeStruct((n, h), jnp.float32),
        ],
    )(x, w1cT)


def _tc2(s1p, degp, r1, b1, w2cT, bias8, n, n_pad, tw):
    def body(s1p_ref, degp_ref, r1_ref, b1_ref, w_ref, b4_ref, t2_ref, inv_ref):
        deg = jnp.maximum(degp_ref[0] + degp_ref[1], 1.0)
        inv = (1.0 / deg)[:, None]
        z = (s1p_ref[0] + s1p_ref[1]) * inv
        hh = jax.nn.relu(z[:n] + r1_ref[...] + b1_ref[...])
        t2 = (
            jnp.dot(hh, w_ref[...], preferred_element_type=jnp.float32)
            + b4_ref[...]
        )
        t2_ref[...] = jnp.concatenate(
            [t2, jnp.zeros((n_pad - n, tw), jnp.float32)])
        inv_ref[...] = inv[:n]

    return pl.pallas_call(
        body,
        out_shape=[
            jax.ShapeDtypeStruct((n_pad, tw), jnp.float32),
            jax.ShapeDtypeStruct((n, 1), jnp.float32),
        ],
    )(s1p, degp, r1, b1, w2cT, bias8)


def _tc3(s2p, t2, inv, n):
    def body(s2p_ref, t2_ref, inv_ref, out_ref):
        s2 = s2p_ref[0, :n, :2] + s2p_ref[1, :n, :2]
        out_ref[...] = s2 * inv_ref[...] + t2_ref[:n, 2:4]

    return pl.pallas_call(
        body,
        out_shape=jax.ShapeDtypeStruct((n, 2), jnp.float32),
    )(s2p, t2, inv)


def kernel(x, edge_index, W1_l, b1, W1_r, W2_l, b2, W2_r):
    n, d = x.shape
    h = W1_l.shape[0]
    o = W2_l.shape[0]
    e = edge_index.shape[1]

    nw = NC * NS
    k1, k2 = 5, 8
    kq = k1 * k2
    step = nw * CH * kq
    e_pad = ((e + step - 1) // step) * step
    chunks = e_pad // (nw * CH)
    n_pad = ((n + 128 * NS - 1) // (128 * NS)) * (128 * NS)

    srcp = jnp.concatenate(
        [edge_index[0], jnp.zeros((e_pad - e,), jnp.int32)]
    ).reshape(nw, chunks, CH)
    pad_dst = n + jnp.arange(e_pad - e, dtype=jnp.int32) % (n_pad - n)
    dstp = jnp.concatenate([edge_index[1], pad_dst]).reshape(nw, chunks, CH)

    w1cT = jnp.concatenate([W1_l, W1_r], axis=0).T
    tw = 8
    w2cT = jnp.concatenate(
        [W2_l, W2_r, jnp.zeros((tw - 2 * o, W2_l.shape[1]), jnp.float32)],
        axis=0).T
    bias8 = jnp.concatenate(
        [jnp.zeros((o,), jnp.float32), b2,
         jnp.zeros((tw - 2 * o,), jnp.float32)])[None, :]

    z32 = jnp.zeros((n_pad, h), jnp.float32)
    z1 = jnp.zeros((n_pad,), jnp.float32)
    z8 = jnp.zeros((n_pad, tw), jnp.float32)

    y1, r1 = _tc1(x, w1cT, n, n_pad, h)
    s1p, degp = _seg_sum_sc(y1, srcp, dstp, z32, z1, n_pad, h, True, k1)
    t2, inv = _tc2(s1p, degp, r1, b1[None, :], w2cT, bias8, n, n_pad, tw)
    s2p = _seg_sum_sc(t2, srcp, dstp, z8, None, n_pad, tw, False, k2)[0]
    out = _tc3(s2p, t2, inv, n)
    return out


# speedup vs baseline: 23.6699x; 1.0859x over previous
import functools

import jax
import jax.numpy as jnp
from jax import lax
from jax.experimental import pallas as pl
from jax.experimental.pallas import tpu as pltpu
from jax.experimental.pallas import tpu_sc as plsc

NC = 2
NS = 16
CH = 128


def _seg_sum_sc(table, eflat, zrows, zdeg, n_pad, width, with_deg, k,
                chunks, e, n):
    nw = NC * NS
    L = chunks * CH
    rl = e - (nw - 1) * L
    rpt = n_pad // NS
    mesh = plsc.VectorSubcoreMesh(core_axis_name="c", subcore_axis_name="s")

    out_type = [jax.ShapeDtypeStruct((NC, n_pad, width), jnp.float32)]
    if with_deg:
        out_type.append(jax.ShapeDtypeStruct((NC, n_pad), jnp.float32))

    scratch = [
        pltpu.VMEM((L,), jnp.int32),
        pltpu.VMEM((L,), jnp.int32),
        pltpu.VMEM((k, CH, width), jnp.float32),
        pltpu.VMEM_SHARED((n_pad, width), jnp.float32),
        pltpu.VMEM_SHARED((n_pad, width), jnp.float32),
        pltpu.SemaphoreType.DMA,
        pltpu.SemaphoreType.DMA,
    ]
    if with_deg:
        scratch += [
            pltpu.VMEM((CH,), jnp.float32),
            pltpu.VMEM_SHARED((n_pad,), jnp.float32),
        ]

    def body(table_h, ef_h, zrows_h, *rest):
        if with_deg:
            (zdeg_h, acc_out, deg_out,
             src_v, dst_v, rows, acc, tbl, sem_g, sem_s, ones, dacc) = rest
        else:
            (acc_out, src_v, dst_v, rows, acc, tbl, sem_g, sem_s) = rest
        c = lax.axis_index("c")
        s = lax.axis_index("s")
        wid = s * NC + c
        off = wid * L
        tile_rows = pl.ds(s * rpt, rpt)
        pltpu.sync_copy(zrows_h.at[tile_rows], acc.at[tile_rows])
        pltpu.sync_copy(table_h.at[tile_rows], tbl.at[tile_rows])

        @pl.when(wid != nw - 1)
        def _():
            pltpu.sync_copy(ef_h.at[pl.ds(off, L)], src_v)
            pltpu.sync_copy(ef_h.at[pl.ds(e + off, L)], dst_v)

        @pl.when(wid == nw - 1)
        def _():
            pltpu.sync_copy(ef_h.at[pl.ds(off, L)], src_v)
            pltpu.sync_copy(ef_h.at[pl.ds(e + off, rl)],
                            dst_v.at[pl.ds(0, rl)])

            def fill(i, carry):
                lane = lax.broadcasted_iota(jnp.int32, (16,), 0)
                dst_v[pl.ds(rl + i * 16, 16)] = (
                    n + (i * 16) % (n_pad - n) + lane)
                return carry

            lax.fori_loop(0, (L - rl) // 16, fill, 0)

        if with_deg:
            pltpu.sync_copy(zdeg_h.at[tile_rows], dacc.at[tile_rows])
            for i in range(CH // 16):
                ones[pl.ds(i * 16, 16)] = jnp.full((16,), 1.0, jnp.float32)
        plsc.subcore_barrier()

        def superchunk(j, carry):
            jk = j * (k * CH)
            gd = [
                pltpu.async_copy(
                    tbl.at[src_v.at[pl.ds(jk + i * CH, CH)]],
                    rows.at[i], sem_g)
                for i in range(k)
            ]
            for d in gd:
                d.wait()
            sd = []
            for i in range(k):
                di = dst_v.at[pl.ds(jk + i * CH, CH)]
                sd.append(pltpu.async_copy(
                    rows.at[i], acc.at[di], sem_s, add=True))
                if with_deg:
                    sd.append(pltpu.async_copy(
                        ones, dacc.at[di], sem_s, add=True))
            for d in sd:
                d.wait()
            return carry

        lax.fori_loop(0, chunks // k, superchunk, 0)
        plsc.subcore_barrier()
        pltpu.sync_copy(acc.at[tile_rows], acc_out.at[c].at[tile_rows])
        if with_deg:
            pltpu.sync_copy(dacc.at[tile_rows], deg_out.at[c].at[tile_rows])

    fn = pl.kernel(
        body, out_type=out_type, mesh=mesh, scratch_types=scratch,
        compiler_params=pltpu.CompilerParams(use_tc_tiling_on_sc=False))
    if with_deg:
        return fn(table, eflat, zrows, zdeg)
    return fn(table, eflat, zrows)


def _tc1(x, w1cT, n, n_pad, h):
    def body(x_ref, w_ref, y1_ref, r1_ref):
        t = jnp.dot(x_ref[...], w_ref[...], preferred_element_type=jnp.float32)
        y1_ref[...] = jnp.concatenate(
            [t[:, :h], jnp.zeros((n_pad - n, h), jnp.float32)])
        r1_ref[...] = t[:, h:]

    return pl.pallas_call(
        body,
        out_shape=[
            jax.ShapeDtypeStruct((n_pad, h), jnp.float32),
            jax.ShapeDtypeStruct((n, h), jnp.float32),
        ],
    )(x, w1cT)


def _tc2(s1p, degp, r1, b1, w2cT, bias8, n, n_pad, tw):
    def body(s1p_ref, degp_ref, r1_ref, b1_ref, w_ref, b4_ref, t2_ref, inv_ref):
        deg = jnp.maximum(degp_ref[0] + degp_ref[1], 1.0)
        inv = (1.0 / deg)[:, None]
        z = (s1p_ref[0] + s1p_ref[1]) * inv
        hh = jax.nn.relu(z[:n] + r1_ref[...] + b1_ref[...])
        t2 = (
            jnp.dot(hh, w_ref[...], preferred_element_type=jnp.float32)
            + b4_ref[...]
        )
        t2_ref[...] = jnp.concatenate(
            [t2, jnp.zeros((n_pad - n, tw), jnp.float32)])
        inv_ref[...] = inv[:n]

    return pl.pallas_call(
        body,
        out_shape=[
            jax.ShapeDtypeStruct((n_pad, tw), jnp.float32),
            jax.ShapeDtypeStruct((n, 1), jnp.float32),
        ],
    )(s1p, degp, r1, b1, w2cT, bias8)


def _tc3(s2p, t2, inv, n):
    def body(s2p_ref, t2_ref, inv_ref, out_ref):
        s2 = s2p_ref[0, :n, :2] + s2p_ref[1, :n, :2]
        out_ref[...] = s2 * inv_ref[...] + t2_ref[:n, 2:4]

    return pl.pallas_call(
        body,
        out_shape=jax.ShapeDtypeStruct((n, 2), jnp.float32),
    )(s2p, t2, inv)


def kernel(x, edge_index, W1_l, b1, W1_r, W2_l, b2, W2_r):
    n, d = x.shape
    h = W1_l.shape[0]
    o = W2_l.shape[0]
    e = edge_index.shape[1]

    nw = NC * NS
    k1, k2 = 5, 8
    kq = k1 * k2
    step = nw * CH * kq
    e_pad = ((e + step - 1) // step) * step
    chunks = e_pad // (nw * CH)
    n_pad = ((n + 128 * NS - 1) // (128 * NS)) * (128 * NS)

    eflat = edge_index.reshape(2 * e)

    w1cT = jnp.concatenate([W1_l, W1_r], axis=0).T
    tw = 8
    w2cT = jnp.concatenate(
        [W2_l, W2_r, jnp.zeros((tw - 2 * o, W2_l.shape[1]), jnp.float32)],
        axis=0).T
    bias8 = jnp.concatenate(
        [jnp.zeros((o,), jnp.float32), b2,
         jnp.zeros((tw - 2 * o,), jnp.float32)])[None, :]

    z32 = jnp.zeros((n_pad, h), jnp.float32)
    z1 = jnp.zeros((n_pad,), jnp.float32)
    z8 = jnp.zeros((n_pad, tw), jnp.float32)

    y1, r1 = _tc1(x, w1cT, n, n_pad, h)
    s1p, degp = _seg_sum_sc(y1, eflat, z32, z1, n_pad, h, True, k1,
                            chunks, e, n)
    t2, inv = _tc2(s1p, degp, r1, b1[None, :], w2cT, bias8, n, n_pad, tw)
    s2p = _seg_sum_sc(t2, eflat, z8, None, n_pad, tw, False, k2,
                      chunks, e, n)[0]
    out = _tc3(s2p, t2, inv, n)
    return out
